# baseline (speedup 1.0000x reference)
import jax
import jax.numpy as jnp
from jax import lax
from jax.experimental import pallas as pl
from jax.experimental.pallas import tpu as pltpu
from jax.experimental.pallas import tpu_sc as plsc

_NF = 26
_NP = 325
_RP = 512
_D = 128
_NC = 2
_NS = 16
_NW = _NC * _NS
_R = _RP // _NW
_L = 16
_NV = _R * _D // _L
_G = 13
_NB = _NP // _G

_PAIRS = [(i, j) for i in range(_NF) for j in range(i + 1, _NF)]


def _sc_body(x_hbm, out_hbm, xv, ob0, ob1, sem):
    wid = lax.axis_index("s") * _NC + lax.axis_index("c")
    r0 = wid * _R
    pltpu.sync_copy(x_hbm.at[:, pl.ds(r0, _R), :], xv)

    bufs = (ob0, ob1)

    def compute_block(buf, block_pairs):
        def vec_step(v, acc):
            r = lax.shift_right_logical(v, 3)
            sl = pl.ds((v & 7) * _L, _L)
            for g, (i, j) in enumerate(block_pairs):
                buf[g, r, sl] = xv[i, r, sl] * xv[j, r, sl]
            return acc

        lax.fori_loop(0, _NV, vec_step, 0)

    for b in range(_NB):
        buf = bufs[b % 2]
        p0 = b * _G
        if b >= 2:
            pltpu.make_async_copy(
                buf,
                out_hbm.at[pl.ds((b - 2) * _G, _G), pl.ds(r0, _R), :],
                sem.at[b % 2],
            ).wait()
        compute_block(buf, _PAIRS[p0:p0 + _G])
        pltpu.async_copy(
            buf,
            out_hbm.at[pl.ds(p0, _G), pl.ds(r0, _R), :],
            sem.at[b % 2],
        )

    for b in (_NB - 2, _NB - 1):
        pltpu.make_async_copy(
            bufs[b % 2],
            out_hbm.at[pl.ds(b * _G, _G), pl.ds(r0, _R), :],
            sem.at[b % 2],
        ).wait()


def kernel(x):
    xr = x.reshape(_NF, _RP, _D)
    k = pl.kernel(
        _sc_body,
        out_type=jax.ShapeDtypeStruct((_NP, _RP, _D), jnp.float32),
        mesh=plsc.VectorSubcoreMesh(core_axis_name="c", subcore_axis_name="s"),
        scratch_types=[
            pltpu.VMEM((_NF, _R, _D), jnp.float32),
            pltpu.VMEM((_G, _R, _D), jnp.float32),
            pltpu.VMEM((_G, _R, _D), jnp.float32),
            pltpu.SemaphoreType.DMA((2,)),
        ],
    )
    out = k(xr)
    return out.reshape(_NP, 1024, 64)


# speedup vs baseline: 1.5164x; 1.5164x over previous
import jax
import jax.numpy as jnp
from jax import lax
from jax.experimental import pallas as pl
from jax.experimental.pallas import tpu as pltpu
from jax.experimental.pallas import tpu_sc as plsc

_NF = 26
_NP = 325
_DT = 64
_BT = 1024
_NC = 2
_NS = 16
_RW = 16
_CW = 128
_L = 16
_NSL = _CW // _L
_GMAX = 12

_CHUNKS = []
_p = 0
for _i in range(_NF - 1):
    _njs = _NF - 1 - _i
    for _o in range(0, _njs, _GMAX):
        _cnt = min(_GMAX, _njs - _o)
        _CHUNKS.append((_i, _i + 1 + _o, _p + _o, _cnt))
    _p += _njs


def _sc_body(x_hbm, out_hbm, xv, ob0, ob1, sem):
    wid = lax.axis_index("s") * _NC + lax.axis_index("c")
    r0 = pl.multiple_of(lax.shift_right_logical(wid, 3) * _RW, _RW)
    c0 = pl.multiple_of((wid & 7) * _CW, _CW)
    pltpu.sync_copy(x_hbm.at[:, pl.ds(r0, _RW), pl.ds(c0, _CW)], xv)

    bufs = (ob0, ob1)

    def dst(chunk):
        _, _, p0, cnt = chunk
        return out_hbm.at[pl.ds(p0, cnt), pl.ds(r0, _RW), pl.ds(c0, _CW)]

    for k, chunk in enumerate(_CHUNKS):
        i, jlo, p0, cnt = chunk
        buf = bufs[k % 2]
        if k >= 2:
            prev = _CHUNKS[k - 2]
            pltpu.make_async_copy(
                buf.at[pl.ds(0, prev[3])], dst(prev), sem.at[k % 2]
            ).wait()

        def r_step(r, acc, buf=buf, i=i, jlo=jlo, cnt=cnt):
            a = [xv[i, r, pl.ds(c * _L, _L)] for c in range(_NSL)]

            def j_step(jj, acc2):
                for c in range(_NSL):
                    sl = pl.ds(c * _L, _L)
                    buf[jj, r, sl] = a[c] * xv[jlo + jj, r, sl]
                return acc2

            lax.fori_loop(0, cnt, j_step, 0)
            return acc

        lax.fori_loop(0, _RW, r_step, 0)

        pltpu.async_copy(buf.at[pl.ds(0, cnt)], dst(chunk), sem.at[k % 2])

    for k in (len(_CHUNKS) - 2, len(_CHUNKS) - 1):
        prev = _CHUNKS[k]
        pltpu.make_async_copy(
            bufs[k % 2].at[pl.ds(0, prev[3])], dst(prev), sem.at[k % 2]
        ).wait()


def kernel(x):
    xt = jnp.transpose(x, (0, 2, 1))
    k = pl.kernel(
        _sc_body,
        out_type=jax.ShapeDtypeStruct((_NP, _DT, _BT), jnp.float32),
        mesh=plsc.VectorSubcoreMesh(core_axis_name="c", subcore_axis_name="s"),
        scratch_types=[
            pltpu.VMEM((_NF, _RW, _CW), jnp.float32),
            pltpu.VMEM((_GMAX, _RW, _CW), jnp.float32),
            pltpu.VMEM((_GMAX, _RW, _CW), jnp.float32),
            pltpu.SemaphoreType.DMA((2,)),
        ],
    )
    out_t = k(xt)
    return jnp.transpose(out_t, (0, 2, 1))
